# baseline (device time: 1706235 ns/iter reference)
import jax
import jax.numpy as jnp
from jax import lax
from jax.experimental import pallas as pl
from jax.experimental.pallas import tpu as pltpu

N_DEV = 32
M, NCOLS = 4096, 8192
HALF = NCOLS // 2
QTR = HALF // 2
CHUNK = M // N_DEV


def _logical_order():
    order = []
    for z in range(4):
        for yi in range(4):
            xs = (0, 1) if yi % 2 == 0 else (1, 0)
            for x in xs:
                order.append((x, yi, z))
    return order


def _ring_order():
    half = []
    for zi in range(4):
        ys = range(4) if zi % 2 == 0 else range(3, -1, -1)
        for y in ys:
            half.append((0, y, zi))
    return half + [(1, y, z) for (_, y, z) in reversed(half)]


_LOG = _logical_order()
_RING = _ring_order()
_L_OF_COORD = {c: l for l, c in enumerate(_LOG)}
_RING_L = [_L_OF_COORD[c] for c in _RING]
_RPOS = [0] * N_DEV
for p, l in enumerate(_RING_L):
    _RPOS[l] = p
_NEXT = [0] * N_DEV
_PREV = [0] * N_DEV
for p, l in enumerate(_RING_L):
    _NEXT[l] = _RING_L[(p + 1) % N_DEV]
    _PREV[l] = _RING_L[(p - 1) % N_DEV]


class _Dir:

    def __init__(self, r, to, fro, col_off, acc, recv, lc,
                 send_sems, recv_sems, lc_sem, out_sem, credit_sems):
        self.r = r
        self.to = to
        self.fro = fro
        self.col_off = col_off
        self.acc = acc
        self.recv = recv
        self.lc = lc
        self.send_sems = send_sems
        self.recv_sems = recv_sems
        self.lc_sem = lc_sem
        self.out_sem = out_sem
        self.credit_sems = credit_sems

    def rdma_sub(self, src, slot, sub):
        return pltpu.make_async_remote_copy(
            src_ref=src,
            dst_ref=self.recv.at[slot, :, pl.ds(sub * QTR, QTR)],
            send_sem=self.send_sems.at[slot, sub],
            recv_sem=self.recv_sems.at[slot, sub],
            device_id=(self.to,),
            device_id_type=pl.DeviceIdType.MESH,
        )

    def rdma_full(self, src, slot):
        return pltpu.make_async_remote_copy(
            src_ref=src,
            dst_ref=self.recv.at[slot],
            send_sem=self.send_sems.at[slot, 0],
            recv_sem=self.recv_sems.at[slot, 0],
            device_id=(self.to,),
            device_id_type=pl.DeviceIdType.MESH,
        )

    def credit(self, sub):
        pl.semaphore_signal(self.credit_sems.at[sub], inc=1,
                            device_id=(self.fro,),
                            device_id_type=pl.DeviceIdType.MESH)

    def credit_wait(self, sub):
        pl.semaphore_wait(self.credit_sems.at[sub], 1)


def _body(part_ref, r_ref, nxt_ref, prv_ref, ringl_ref, out_ref,
          acc_f, recv_f, lc_f, acc_b, recv_b, lc_b,
          send_f, recv_sf, lc_sf, out_sf, cred_f,
          send_b, recv_sb, lc_sb, out_sb, cred_b,
          sbuf, rbuf, bsend, brecv):
    r = r_ref[0]
    nxt = nxt_ref[0]
    prv = prv_ref[0]
    r_b = (N_DEV - r) % N_DEV

    F = _Dir(r, nxt, prv, 0, acc_f, recv_f, lc_f,
             send_f, recv_sf, lc_sf, out_sf, cred_f)
    B = _Dir(r_b, prv, nxt, HALF, acc_b, recv_b, lc_b,
             send_b, recv_sb, lc_sb, out_sb, cred_b)
    dirs = (F, B)
    SUBS = (0, 1)

    def part_slice(d, c):
        return part_ref.at[pl.ds(c * CHUNK, CHUNK), pl.ds(d.col_off, HALF)]

    pres, lds = [], []
    for d in dirs:
        pre = pltpu.make_async_copy(part_slice(d, (d.r - 1) % N_DEV),
                                    d.acc.at[0], d.out_sem)
        pre.start()
        pres.append(pre)
        ld = pltpu.make_async_copy(part_slice(d, (d.r - 2) % N_DEV),
                                   d.lc.at[0], d.lc_sem)
        ld.start()
        lds.append(ld)

    barrier_sem = pltpu.get_barrier_semaphore()
    pl.semaphore_signal(barrier_sem, inc=1, device_id=(nxt,),
                        device_id_type=pl.DeviceIdType.MESH)
    pl.semaphore_signal(barrier_sem, inc=1, device_id=(prv,),
                        device_id_type=pl.DeviceIdType.MESH)
    pl.semaphore_wait(barrier_sem, 2)

    rd = {}
    for d, pre in zip(dirs, pres):
        pre.wait()
        for sub in SUBS:
            rm = d.rdma_sub(d.acc.at[0, :, pl.ds(sub * QTR, QTR)], 0, sub)
            rm.start()
            rd[(id(d), sub)] = rm
    ld_cur = lds

    for s in range(N_DEV - 1):
        slot = s % 2
        nslot = 1 - slot
        ld_next = []
        for di, d in enumerate(dirs):
            for sub in SUBS:
                cols = pl.ds(sub * QTR, QTR)
                rd[(id(d), sub)].wait()
                if sub == 0:
                    ld_cur[di].wait()
                if s < N_DEV - 2:
                    d.acc[nslot, :, sub * QTR:(sub + 1) * QTR] = (
                        d.recv[slot, :, sub * QTR:(sub + 1) * QTR]
                        + d.lc[slot, :, sub * QTR:(sub + 1) * QTR])
                else:
                    d.acc[nslot, :, sub * QTR:(sub + 1) * QTR] = jnp.maximum(
                        d.recv[slot, :, sub * QTR:(sub + 1) * QTR]
                        + d.lc[slot, :, sub * QTR:(sub + 1) * QTR], 0.0)
                d.credit(sub)
                if s < N_DEV - 2:
                    if s >= 1:
                        d.credit_wait(sub)
                    rm = d.rdma_sub(d.acc.at[nslot, :, cols], nslot, sub)
                    rm.start()
                    rd[(id(d), sub)] = rm
                if sub == 1 and s < N_DEV - 2:
                    ld = pltpu.make_async_copy(
                        part_slice(d, (d.r - 3 - s) % N_DEV),
                        d.lc.at[nslot], d.lc_sem)
                    ld.start()
                    ld_next.append(ld)
        ld_cur = ld_next

    val = jnp.maximum(jnp.max(F.acc[1]), jnp.max(B.acc[1]))
    for k in range(5):
        partner = ringl_ref[r ^ (1 << k)]
        sbuf[...] = jnp.zeros((8, 128), jnp.float32) + val
        ex = pltpu.make_async_remote_copy(
            src_ref=sbuf, dst_ref=rbuf.at[k],
            send_sem=bsend.at[k], recv_sem=brecv.at[k],
            device_id=(partner,), device_id_type=pl.DeviceIdType.MESH)
        ex.start()
        ex.wait()
        val = jnp.maximum(val, jnp.max(rbuf[k]))

    scale = val / 448.0

    def dq(x):
        v = jnp.minimum(x / scale, 448.0)
        bits = lax.bitcast_convert_type(v, jnp.int32)
        rb = (bits + 0x7FFFF + ((bits >> 20) & 1)) & ~0xFFFFF
        normal = lax.bitcast_convert_type(rb, jnp.float32)
        subn = jnp.round(v * 512.0) * jnp.float32(1.0 / 512.0)
        return jnp.where(v < 2.0 ** -6, subn, normal) * scale

    for t in range(N_DEV - 1):
        slot = (N_DEV - 1 + t) % 2
        rdmas = []
        if t == 0:
            for d in dirs:
                d.acc[1] = dq(d.acc[1])
                for sub in SUBS:
                    d.credit_wait(sub)
                rm = d.rdma_full(d.acc.at[1], slot)
                rm.start()
                rdmas.append(rm)
        else:
            for d in dirs:
                for sub in SUBS:
                    d.credit_wait(sub)
            for d in dirs:
                rm = d.rdma_full(d.recv.at[1 - slot], slot)
                rm.start()
                rdmas.append(rm)
        sts = []
        for d in dirs:
            if t == 0:
                src, row = d.acc.at[1], d.r * CHUNK
            else:
                origin = (d.r - t) % N_DEV
                src, row = d.recv.at[1 - slot], origin * CHUNK
            st = pltpu.make_async_copy(
                src, out_ref.at[pl.ds(row, CHUNK), pl.ds(d.col_off, HALF)],
                d.out_sem)
            st.start()
            sts.append(st)
        for st in sts:
            st.wait()
        for d, rm in zip(dirs, rdmas):
            rm.wait()
            if t > 0:
                for sub in SUBS:
                    d.credit(sub)
    slot = (2 * N_DEV - 3) % 2
    sts = []
    for d in dirs:
        origin = (d.r - (N_DEV - 1)) % N_DEV
        st = pltpu.make_async_copy(
            d.recv.at[slot],
            out_ref.at[pl.ds(origin * CHUNK, CHUNK), pl.ds(d.col_off, HALF)],
            d.out_sem)
        st.start()
        sts.append(st)
    for st, d in zip(sts, dirs):
        st.wait()
        for sub in SUBS:
            d.credit(sub)

    for d in dirs:
        for sub in SUBS:
            pl.semaphore_wait(d.credit_sems.at[sub], 2)


def _all_reduce_relu_quant(partial, r, nxt, prv, ringl):
    dir_scratch = [
        pltpu.VMEM((2, CHUNK, HALF), jnp.float32),
        pltpu.VMEM((2, CHUNK, HALF), jnp.float32),
        pltpu.VMEM((2, CHUNK, HALF), jnp.float32),
    ]
    dir_sems = [
        pltpu.SemaphoreType.DMA((2, 2)),
        pltpu.SemaphoreType.DMA((2, 2)),
        pltpu.SemaphoreType.DMA,
        pltpu.SemaphoreType.DMA,
        pltpu.SemaphoreType.REGULAR((2,)),
    ]
    butterfly = [
        pltpu.VMEM((8, 128), jnp.float32),
        pltpu.VMEM((5, 8, 128), jnp.float32),
        pltpu.SemaphoreType.DMA((5,)),
        pltpu.SemaphoreType.DMA((5,)),
    ]
    return pl.pallas_call(
        _body,
        out_shape=jax.ShapeDtypeStruct((M, NCOLS), jnp.float32),
        in_specs=[
            pl.BlockSpec(memory_space=pl.ANY),
            pl.BlockSpec(memory_space=pltpu.SMEM),
            pl.BlockSpec(memory_space=pltpu.SMEM),
            pl.BlockSpec(memory_space=pltpu.SMEM),
            pl.BlockSpec(memory_space=pltpu.SMEM),
        ],
        out_specs=pl.BlockSpec(memory_space=pl.ANY),
        scratch_shapes=(dir_scratch + dir_scratch
                        + dir_sems + dir_sems + butterfly),
        compiler_params=pltpu.CompilerParams(collective_id=0),
    )(partial, r, nxt, prv, ringl)


def kernel(x, w_mat):
    partial = lax.dot_general(
        x, w_mat, (((1,), (0,)), ((), ())),
        precision=lax.Precision.HIGHEST,
        preferred_element_type=jnp.float32,
    )
    i = lax.axis_index("i")
    r = jnp.asarray(_RPOS, jnp.int32)[i].reshape(1)
    nxt = jnp.asarray(_NEXT, jnp.int32)[i].reshape(1)
    prv = jnp.asarray(_PREV, jnp.int32)[i].reshape(1)
    ringl = jnp.asarray(_RING_L, jnp.int32)
    return _all_reduce_relu_quant(partial, r, nxt, prv, ringl)


# device time: 1599921 ns/iter; 1.0664x vs baseline; 1.0664x over previous
import jax
import jax.numpy as jnp
from jax import lax
from jax.experimental import pallas as pl
from jax.experimental.pallas import tpu as pltpu

N_DEV = 32
M, NCOLS = 4096, 8192
HALF = NCOLS // 2
QTR = HALF // 2
CHUNK = M // N_DEV


def _logical_order():
    order = []
    for z in range(4):
        for yi in range(4):
            xs = (0, 1) if yi % 2 == 0 else (1, 0)
            for x in xs:
                order.append((x, yi, z))
    return order


def _ring_order():
    half = []
    for zi in range(4):
        ys = range(4) if zi % 2 == 0 else range(3, -1, -1)
        for y in ys:
            half.append((0, y, zi))
    return half + [(1, y, z) for (_, y, z) in reversed(half)]


_LOG = _logical_order()
_RING = _ring_order()
_L_OF_COORD = {c: l for l, c in enumerate(_LOG)}
_RING_L = [_L_OF_COORD[c] for c in _RING]
_RPOS = [0] * N_DEV
for p, l in enumerate(_RING_L):
    _RPOS[l] = p
_NEXT = [0] * N_DEV
_PREV = [0] * N_DEV
for p, l in enumerate(_RING_L):
    _NEXT[l] = _RING_L[(p + 1) % N_DEV]
    _PREV[l] = _RING_L[(p - 1) % N_DEV]


class _Dir:

    def __init__(self, r, to, fro, col_off, acc, recv,
                 send_sems, recv_sems, out_sem, credit_sems):
        self.r = r
        self.to = to
        self.fro = fro
        self.col_off = col_off
        self.acc = acc
        self.recv = recv
        self.send_sems = send_sems
        self.recv_sems = recv_sems
        self.out_sem = out_sem
        self.credit_sems = credit_sems

    def rdma_sub(self, src, slot, sub):
        return pltpu.make_async_remote_copy(
            src_ref=src,
            dst_ref=self.recv.at[slot, :, pl.ds(sub * QTR, QTR)],
            send_sem=self.send_sems.at[slot, sub],
            recv_sem=self.recv_sems.at[slot, sub],
            device_id=(self.to,),
            device_id_type=pl.DeviceIdType.MESH,
        )

    def rdma_full(self, src, slot):
        return pltpu.make_async_remote_copy(
            src_ref=src,
            dst_ref=self.recv.at[slot],
            send_sem=self.send_sems.at[slot, 0],
            recv_sem=self.recv_sems.at[slot, 0],
            device_id=(self.to,),
            device_id_type=pl.DeviceIdType.MESH,
        )

    def credit(self, sub):
        pl.semaphore_signal(self.credit_sems.at[sub], inc=1,
                            device_id=(self.fro,),
                            device_id_type=pl.DeviceIdType.MESH)

    def credit_wait(self, sub):
        pl.semaphore_wait(self.credit_sems.at[sub], 1)


def _body(x_ref, w_ref, r_ref, nxt_ref, prv_ref, ringl_ref, out_ref,
          acc_f, recv_f, acc_b, recv_b,
          send_f, recv_sf, out_sf, cred_f,
          send_b, recv_sb, out_sb, cred_b,
          sbuf, rbuf, bsend, brecv):
    r = r_ref[0]
    nxt = nxt_ref[0]
    prv = prv_ref[0]
    r_b = (N_DEV - r) % N_DEV

    F = _Dir(r, nxt, prv, 0, acc_f, recv_f,
             send_f, recv_sf, out_sf, cred_f)
    B = _Dir(r_b, prv, nxt, HALF, acc_b, recv_b,
             send_b, recv_sb, out_sb, cred_b)
    dirs = (F, B)
    SUBS = (0, 1)

    def mm(d, c):
        return lax.dot_general(
            x_ref[pl.ds(c * CHUNK, CHUNK), :],
            w_ref[:, d.col_off:d.col_off + HALF],
            (((1,), (0,)), ((), ())),
            precision=lax.Precision.HIGHEST,
            preferred_element_type=jnp.float32,
        )

    for d in dirs:
        d.acc[0] = mm(d, (d.r - 1) % N_DEV)

    barrier_sem = pltpu.get_barrier_semaphore()
    pl.semaphore_signal(barrier_sem, inc=1, device_id=(nxt,),
                        device_id_type=pl.DeviceIdType.MESH)
    pl.semaphore_signal(barrier_sem, inc=1, device_id=(prv,),
                        device_id_type=pl.DeviceIdType.MESH)
    pl.semaphore_wait(barrier_sem, 2)

    rd = {}
    for d in dirs:
        for sub in SUBS:
            rm = d.rdma_sub(d.acc.at[0, :, pl.ds(sub * QTR, QTR)], 0, sub)
            rm.start()
            rd[(id(d), sub)] = rm

    for s in range(N_DEV - 1):
        slot = s % 2
        nslot = 1 - slot
        for d in dirs:
            tmp = mm(d, (d.r - 2 - s) % N_DEV)
            for sub in SUBS:
                cols = pl.ds(sub * QTR, QTR)
                rd[(id(d), sub)].wait()
                if s < N_DEV - 2:
                    d.acc[nslot, :, sub * QTR:(sub + 1) * QTR] = (
                        d.recv[slot, :, sub * QTR:(sub + 1) * QTR]
                        + tmp[:, sub * QTR:(sub + 1) * QTR])
                else:
                    d.acc[nslot, :, sub * QTR:(sub + 1) * QTR] = jnp.maximum(
                        d.recv[slot, :, sub * QTR:(sub + 1) * QTR]
                        + tmp[:, sub * QTR:(sub + 1) * QTR], 0.0)
                d.credit(sub)
                if s < N_DEV - 2:
                    if s >= 1:
                        d.credit_wait(sub)
                    rm = d.rdma_sub(d.acc.at[nslot, :, cols], nslot, sub)
                    rm.start()
                    rd[(id(d), sub)] = rm

    val = jnp.maximum(jnp.max(F.acc[1]), jnp.max(B.acc[1]))
    for k in range(5):
        partner = ringl_ref[r ^ (1 << k)]
        sbuf[...] = jnp.zeros((8, 128), jnp.float32) + val
        ex = pltpu.make_async_remote_copy(
            src_ref=sbuf, dst_ref=rbuf.at[k],
            send_sem=bsend.at[k], recv_sem=brecv.at[k],
            device_id=(partner,), device_id_type=pl.DeviceIdType.MESH)
        ex.start()
        ex.wait()
        val = jnp.maximum(val, jnp.max(rbuf[k]))

    scale = val / 448.0

    def dq(x):
        v = jnp.minimum(x / scale, 448.0)
        bits = lax.bitcast_convert_type(v, jnp.int32)
        rb = (bits + 0x7FFFF + ((bits >> 20) & 1)) & ~0xFFFFF
        normal = lax.bitcast_convert_type(rb, jnp.float32)
        subn = jnp.round(v * 512.0) * jnp.float32(1.0 / 512.0)
        return jnp.where(v < 2.0 ** -6, subn, normal) * scale

    for t in range(N_DEV - 1):
        slot = (N_DEV - 1 + t) % 2
        rdmas = []
        if t == 0:
            for d in dirs:
                d.acc[1] = dq(d.acc[1])
                for sub in SUBS:
                    d.credit_wait(sub)
                rm = d.rdma_full(d.acc.at[1], slot)
                rm.start()
                rdmas.append(rm)
        else:
            for d in dirs:
                for sub in SUBS:
                    d.credit_wait(sub)
            for d in dirs:
                rm = d.rdma_full(d.recv.at[1 - slot], slot)
                rm.start()
                rdmas.append(rm)
        sts = []
        for d in dirs:
            if t == 0:
                src, row = d.acc.at[1], d.r * CHUNK
            else:
                origin = (d.r - t) % N_DEV
                src, row = d.recv.at[1 - slot], origin * CHUNK
            st = pltpu.make_async_copy(
                src, out_ref.at[pl.ds(row, CHUNK), pl.ds(d.col_off, HALF)],
                d.out_sem)
            st.start()
            sts.append(st)
        for st in sts:
            st.wait()
        for d, rm in zip(dirs, rdmas):
            rm.wait()
            if t > 0:
                for sub in SUBS:
                    d.credit(sub)
    slot = (2 * N_DEV - 3) % 2
    sts = []
    for d in dirs:
        origin = (d.r - (N_DEV - 1)) % N_DEV
        st = pltpu.make_async_copy(
            d.recv.at[slot],
            out_ref.at[pl.ds(origin * CHUNK, CHUNK), pl.ds(d.col_off, HALF)],
            d.out_sem)
        st.start()
        sts.append(st)
    for st, d in zip(sts, dirs):
        st.wait()
        for sub in SUBS:
            d.credit(sub)

    for d in dirs:
        for sub in SUBS:
            pl.semaphore_wait(d.credit_sems.at[sub], 2)


def _all_reduce_relu_quant(x, w_mat, r, nxt, prv, ringl):
    dir_scratch = [
        pltpu.VMEM((2, CHUNK, HALF), jnp.float32),
        pltpu.VMEM((2, CHUNK, HALF), jnp.float32),
    ]
    dir_sems = [
        pltpu.SemaphoreType.DMA((2, 2)),
        pltpu.SemaphoreType.DMA((2, 2)),
        pltpu.SemaphoreType.DMA,
        pltpu.SemaphoreType.REGULAR((2,)),
    ]
    butterfly = [
        pltpu.VMEM((8, 128), jnp.float32),
        pltpu.VMEM((5, 8, 128), jnp.float32),
        pltpu.SemaphoreType.DMA((5,)),
        pltpu.SemaphoreType.DMA((5,)),
    ]
    return pl.pallas_call(
        _body,
        out_shape=jax.ShapeDtypeStruct((M, NCOLS), jnp.float32),
        in_specs=[
            pl.BlockSpec(memory_space=pltpu.VMEM),
            pl.BlockSpec(memory_space=pltpu.VMEM),
            pl.BlockSpec(memory_space=pltpu.SMEM),
            pl.BlockSpec(memory_space=pltpu.SMEM),
            pl.BlockSpec(memory_space=pltpu.SMEM),
            pl.BlockSpec(memory_space=pltpu.SMEM),
        ],
        out_specs=pl.BlockSpec(memory_space=pl.ANY),
        scratch_shapes=(dir_scratch + dir_scratch
                        + dir_sems + dir_sems + butterfly),
        compiler_params=pltpu.CompilerParams(collective_id=0),
    )(x, w_mat, r, nxt, prv, ringl)


def kernel(x, w_mat):
    i = lax.axis_index("i")
    r = jnp.asarray(_RPOS, jnp.int32)[i].reshape(1)
    nxt = jnp.asarray(_NEXT, jnp.int32)[i].reshape(1)
    prv = jnp.asarray(_PREV, jnp.int32)[i].reshape(1)
    ringl = jnp.asarray(_RING_L, jnp.int32)
    return _all_reduce_relu_quant(x, w_mat, r, nxt, prv, ringl)
